# Initial kernel scaffold; baseline (speedup 1.0000x reference)
import jax
import jax.numpy as jnp
from jax.experimental import pallas as pl


def kernel(x, pos_orig, agg_weight, idx_agg, H, W, N_grid, grid_merge, conv_w, conv_b, skip_w, ln_w, ln_b, conf_w, conf_b):
    raise NotImplementedError("write your pallas kernel here")


# speedup vs baseline: 1.8691x; 1.8691x over previous
import functools

import jax
import jax.numpy as jnp
from jax import lax
from jax.experimental import pallas as pl
from jax.experimental.pallas import tpu as pltpu
from jax.experimental.pallas import tpu_sc as plsc

B = 16
N = 12544
C_IN = 64
C_OUT = 128
H0 = 112
W0 = 112
HM = 56
WM = 56
NM = HM * WM

NCORE = 2
NSUB = 16
LANES = 16
TPB = N // NSUB
CH = 56
NCH = TPB // CH
QPC = CH // 8
RWT = TPB * LANES // 128


def _conv_body(sp_ref, pos_ref, w_ref, b_ref, xmap_ref, pix_ref):
    sp = sp_ref[0]
    acc = jnp.zeros((NM, C_OUT), jnp.float32)
    k = 0
    for di in range(2):
        for dj in range(2):
            xs = lax.slice(sp, (di, dj, 0), (di + HM, dj + WM, 4 * C_IN))
            acc = acc + jnp.dot(xs.reshape(NM, 4 * C_IN), w_ref[k],
                                preferred_element_type=jnp.float32)
            k += 1
    xmap_ref[0] = acc + b_ref[0]

    pos = pos_ref[0]
    lx = 0.5 * (pos[0:1, :] + 1.0) * WM - 0.5
    ly = 0.5 * (pos[1:2, :] + 1.0) * HM - 0.5
    xi = jnp.clip(jnp.round(lx), 0, WM - 1).astype(jnp.int32)
    yi = jnp.clip(jnp.round(ly), 0, HM - 1).astype(jnp.int32)
    pix_ref[0] = pl.program_id(0) * NM + yi * WM + xi


def _conv_call(sp, pos_t, w4, b2):
    return pl.pallas_call(
        _conv_body,
        grid=(B,),
        in_specs=[
            pl.BlockSpec((1, HM + 1, WM + 1, 4 * C_IN), lambda b: (b, 0, 0, 0)),
            pl.BlockSpec((1, 2, N), lambda b: (b, 0, 0)),
            pl.BlockSpec((4, 4 * C_IN, C_OUT), lambda b: (0, 0, 0)),
            pl.BlockSpec((1, C_OUT), lambda b: (0, 0)),
        ],
        out_specs=[
            pl.BlockSpec((1, NM, C_OUT), lambda b: (b, 0, 0)),
            pl.BlockSpec((1, 1, N), lambda b: (b, 0, 0)),
        ],
        out_shape=[
            jax.ShapeDtypeStruct((B, NM, C_OUT), jnp.float32),
            jax.ShapeDtypeStruct((B, 1, N), jnp.int32),
        ],
    )(sp, pos_t, w4, b2)


def _sc_body(xmap_hbm, pix_hbm, agg_hbm, wrep_hbm,
             tok_out,
             pix_v, agg_v, wrep_c, rows_v, wpay_v,
             tok_sh, allw_sh):
    cid = lax.axis_index("c")
    sid = lax.axis_index("s")
    tok_base = sid * TPB
    zf = jnp.zeros((LANES,), jnp.float32)

    def batch_body(b_local, carry):
        b = cid * (B // NCORE) + b_local
        slot = b * NSUB + sid

        def zfill(r, c):
            for cc in range(C_OUT // LANES):
                rows_v[r, pl.ds(cc * LANES, LANES)] = zf
            wpay_v[r, :] = zf
            return c
        lax.fori_loop(0, CH, zfill, 0)

        for j in range(NCH):
            pltpu.sync_copy(rows_v, tok_sh.at[pl.ds(tok_base + j * CH, CH)])
            pltpu.sync_copy(wpay_v, allw_sh.at[pl.ds(tok_base + j * CH, CH)])

        plsc.subcore_barrier()

        def chunk_body(j, c):
            pltpu.sync_copy(pix_hbm.at[slot * NCH + j], pix_v)
            pltpu.sync_copy(agg_hbm.at[slot * NCH + j, 0], agg_v)
            pltpu.sync_copy(xmap_hbm.at[pix_v.at[0]], rows_v)
            pltpu.sync_copy(wrep_hbm.at[slot * NCH + j], wrep_c)

            def q_body(q, c2):
                for m in range(8):
                    r = q * 8 + m
                    wv = wrep_c[q, pl.ds(m * LANES, LANES)]
                    for cc in range(C_OUT // LANES):
                        rows_v[r, pl.ds(cc * LANES, LANES)] = (
                            rows_v[r, pl.ds(cc * LANES, LANES)] * wv)
                return c2
            lax.fori_loop(0, QPC, q_body, 0)

            pltpu.sync_copy(rows_v, tok_sh.at[agg_v], add=True)
            return c
        lax.fori_loop(0, NCH, chunk_body, 0)

        def wchunk_body(j, c):
            pltpu.sync_copy(agg_hbm.at[slot * NCH + j, 0], agg_v)
            pltpu.sync_copy(wrep_hbm.at[slot * NCH + j], wrep_c)

            def qw_body(q, c2):
                for m in range(8):
                    r = q * 8 + m
                    wpay_v[r, :] = wrep_c[q, pl.ds(m * LANES, LANES)]
                return c2
            lax.fori_loop(0, QPC, qw_body, 0)

            pltpu.sync_copy(wpay_v, allw_sh.at[agg_v], add=True)
            return c
        lax.fori_loop(0, NCH, wchunk_body, 0)

        plsc.subcore_barrier()

        def qd_body(q, c2):
            for m in range(8):
                r = q * 8 + m
                rv = 1.0 / (wpay_v[r, :] + 1e-6)
                for cc in range(C_OUT // LANES):
                    rows_v[r, pl.ds(cc * LANES, LANES)] = (
                        rows_v[r, pl.ds(cc * LANES, LANES)] * rv)
            return c2

        for j in range(NCH):
            start = tok_base + j * CH
            pltpu.sync_copy(tok_sh.at[pl.ds(start, CH)], rows_v)
            pltpu.sync_copy(allw_sh.at[pl.ds(start, CH)], wpay_v)
            lax.fori_loop(0, QPC, qd_body, 0)
            pltpu.sync_copy(rows_v, tok_out.at[b, pl.ds(start, CH)])
        return carry

    lax.fori_loop(0, B // NCORE, batch_body, 0)


def _sc_call(x_map, pix3, agg3, wrep4):
    mesh = plsc.VectorSubcoreMesh(core_axis_name="c", subcore_axis_name="s")
    f = functools.partial(
        pl.kernel,
        mesh=mesh,
        out_type=jax.ShapeDtypeStruct((B, N, C_OUT), jnp.float32),
        scratch_types=[
            pltpu.VMEM((1, CH), jnp.int32),
            pltpu.VMEM((CH,), jnp.int32),
            pltpu.VMEM((8, 128), jnp.float32),
            pltpu.VMEM((CH, C_OUT), jnp.float32),
            pltpu.VMEM((CH, LANES), jnp.float32),
            pltpu.VMEM_SHARED((N, C_OUT), jnp.float32),
            pltpu.VMEM_SHARED((N, LANES), jnp.float32),
        ],
    )(_sc_body)
    return f(x_map, pix3, agg3, wrep4)


NS = 7
NT = N // NS
NR = H0 // (2 * NS)


def _post_body(tok_ref, x_ref, aw_ref, skw_ref, lnw_ref, lnb_ref,
               cfw_ref, cfb_ref, xd_ref, awd_ref):
    t = tok_ref[0]
    y = t + jnp.dot(x_ref[0], skw_ref[...], preferred_element_type=jnp.float32)
    m = jnp.mean(y, axis=-1, keepdims=True)
    yc = y - m
    v = jnp.mean(yc * yc, axis=-1, keepdims=True)
    x2 = yc / jnp.sqrt(v + 1e-5) * lnw_ref[0] + lnb_ref[0]
    conf = jnp.sum(x2 * cfw_ref[0], axis=-1, keepdims=True) + cfb_ref[0, 0]
    wgt = jnp.exp(conf)
    w4 = wgt.reshape(NR, 2, WM, 2, 1)
    m4 = jnp.mean(w4, axis=(1, 3), keepdims=True)
    nw4 = w4 / (m4 + 1e-6)
    x24 = x2.reshape(NR, 2, WM, 2, C_OUT)
    xd = jnp.mean(x24 * nw4, axis=(1, 3))
    xd_ref[0] = xd.reshape(NT // 4, C_OUT)
    nwf = nw4.reshape(1, NT)
    awd_ref[0] = aw_ref[0] * (nwf * 0.25)


def _post_call(tok, x, aw2, skw, lnw, lnb, cfw, cfb):
    return pl.pallas_call(
        _post_body,
        grid=(B, NS),
        in_specs=[
            pl.BlockSpec((1, NT, C_OUT), lambda b, s: (b, s, 0)),
            pl.BlockSpec((1, NT, C_IN), lambda b, s: (b, s, 0)),
            pl.BlockSpec((1, 1, NT), lambda b, s: (b, 0, s)),
            pl.BlockSpec((C_IN, C_OUT), lambda b, s: (0, 0)),
            pl.BlockSpec((1, C_OUT), lambda b, s: (0, 0)),
            pl.BlockSpec((1, C_OUT), lambda b, s: (0, 0)),
            pl.BlockSpec((1, C_OUT), lambda b, s: (0, 0)),
            pl.BlockSpec((1, 1), lambda b, s: (0, 0)),
        ],
        out_specs=[
            pl.BlockSpec((1, NT // 4, C_OUT), lambda b, s: (b, s, 0)),
            pl.BlockSpec((1, 1, NT), lambda b, s: (b, 0, s)),
        ],
        out_shape=[
            jax.ShapeDtypeStruct((B, NM, C_OUT), jnp.float32),
            jax.ShapeDtypeStruct((B, 1, N), jnp.float32),
        ],
    )(tok, x, aw2, skw, lnw, lnb, cfw, cfb)


def _norm_body(awt_ref, awd_ref):
    a = awt_ref[0]
    mx = jnp.max(a, axis=1, keepdims=True)
    awd_ref[0] = a / mx


def _norm_call(awd_t):
    return pl.pallas_call(
        _norm_body,
        grid=(B,),
        in_specs=[pl.BlockSpec((1, 1, N), lambda b: (b, 0, 0))],
        out_specs=pl.BlockSpec((1, 1, N), lambda b: (b, 0, 0)),
        out_shape=jax.ShapeDtypeStruct((B, 1, N), jnp.float32),
    )(awd_t)


def _shifted_weights(conv_w):
    w4 = jnp.zeros((4, 2, 2, C_IN, C_OUT), jnp.float32)
    taps = {
        0: [((1, 1), (0, 0))],
        1: [((1, 0), (0, 1)), ((1, 1), (0, 2))],
        2: [((0, 1), (1, 0)), ((1, 1), (2, 0))],
        3: [((0, 0), (1, 1)), ((0, 1), (1, 2)),
            ((1, 0), (2, 1)), ((1, 1), (2, 2))],
    }
    for k, lst in taps.items():
        for (p, q), (dy, dx) in lst:
            w4 = w4.at[k, p, q].set(conv_w[dy, dx])
    return w4.reshape(4, 4 * C_IN, C_OUT)


def kernel(x, pos_orig, agg_weight, idx_agg, H, W, N_grid, grid_merge,
           conv_w, conv_b, skip_w, ln_w, ln_b, conf_w, conf_b):
    s2d = (x.reshape(B, HM, 2, WM, 2, C_IN)
           .transpose(0, 1, 3, 2, 4, 5)
           .reshape(B, HM, WM, 4 * C_IN))
    sp = jnp.pad(s2d, ((0, 0), (1, 0), (1, 0), (0, 0)))
    pos_t = pos_orig.transpose(0, 2, 1)
    w4 = _shifted_weights(conv_w)
    x_map, pix = _conv_call(sp, pos_t, w4, conv_b.reshape(1, C_OUT))

    pixf = pix.reshape(B, N)
    w = agg_weight.reshape(B, N)
    gathered = jnp.take(x_map.reshape(B * NM, C_OUT), pixf, axis=0)
    seg = (idx_agg.astype(jnp.int32)
           + jnp.arange(B, dtype=jnp.int32)[:, None] * N).reshape(-1)
    tok_raw = jax.ops.segment_sum(
        (gathered * w[..., None]).reshape(B * N, C_OUT), seg,
        num_segments=B * N)
    allw = jax.ops.segment_sum(w.reshape(-1), seg, num_segments=B * N)
    tok = (tok_raw / (allw[:, None] + 1e-6)).reshape(B, N, C_OUT)

    xd, awd_t = _post_call(tok, x, agg_weight.reshape(B, 1, N),
                           skip_w.T, ln_w.reshape(1, -1), ln_b.reshape(1, -1),
                           conf_w.reshape(1, -1), conf_b.reshape(1, 1))
    awd = _norm_call(awd_t)

    idx = jnp.arange(NM).reshape(HM, WM)
    idx_down = (jnp.repeat(jnp.repeat(idx, 2, axis=0), 2, axis=1)
                .reshape(-1)[None, :])
    idx_agg_down = jnp.tile(idx_down, (B, 1))
    return xd, idx_agg_down, awd.reshape(B, N, 1)
